# Initial kernel scaffold; baseline (speedup 1.0000x reference)
import jax
import jax.numpy as jnp
from jax.experimental import pallas as pl


def kernel(sentences, rules):
    raise NotImplementedError("write your pallas kernel here")


# speedup vs baseline: 3.4354x; 3.4354x over previous
import functools

import jax
import jax.numpy as jnp
from jax import lax
from jax.experimental import pallas as pl
from jax.experimental.pallas import tpu as pltpu
from jax.experimental.pallas import tpu_sc as plsc

_NUM_PT = 32
_NUM_T = 100000
_BATCH = 1024
_SEQ = 200
_SEQ_PAD = 208
_NB = 32
_NCHUNK = _BATCH // _NB
_NVEC = _SEQ_PAD // 16

_mesh = plsc.VectorSubcoreMesh(core_axis_name="c", subcore_axis_name="s")


@functools.partial(
    pl.kernel,
    mesh=_mesh,
    compiler_params=pltpu.CompilerParams(use_tc_tiling_on_sc=False,
                                          needs_layout_passes=False),
    out_type=jax.ShapeDtypeStruct((_BATCH, _NUM_PT, _SEQ), jnp.float32),
    scratch_types=[
        pltpu.VMEM((_NUM_T,), jnp.float32),
        pltpu.VMEM((_NB, _SEQ_PAD), jnp.int32),
        pltpu.VMEM((_NB, _SEQ_PAD), jnp.float32),
    ],
)
def _sc_lookup(sent_hbm, rules_hbm, out_hbm, row_v, idx_v, outbuf_v):
    wid = lax.axis_index("s") * 2 + lax.axis_index("c")
    pltpu.sync_copy(rules_hbm.at[wid], row_v)

    def chunk_body(ci, _):
        b0 = ci * _NB
        pltpu.sync_copy(sent_hbm.at[pl.ds(b0, _NB)], idx_v)

        def sent_body(s, _):
            for j in range(_NVEC):
                idx = idx_v[s, pl.ds(j * 16, 16)]
                outbuf_v[s, pl.ds(j * 16, 16)] = plsc.load_gather(row_v, [idx])
            return 0

        lax.fori_loop(0, _NB, sent_body, 0)
        pltpu.sync_copy(outbuf_v.at[:, pl.ds(0, _SEQ)],
                        out_hbm.at[pl.ds(b0, _NB), wid])
        return 0

    lax.fori_loop(0, _NCHUNK, chunk_body, 0)


def kernel(sentences, rules):
    sent_pad = jnp.pad(sentences.astype(jnp.int32),
                       ((0, 0), (0, _SEQ_PAD - _SEQ)))
    return _sc_lookup(sent_pad, rules)


# speedup vs baseline: 4.0845x; 1.1890x over previous
import functools

import jax
import jax.numpy as jnp
from jax import lax
from jax.experimental import pallas as pl
from jax.experimental.pallas import tpu as pltpu
from jax.experimental.pallas import tpu_sc as plsc

_NUM_PT = 32
_NUM_T = 100000
_BATCH = 1024
_SEQ = 200
_SEQ_PAD = 208
_NB = 32
_NCHUNK = _BATCH // _NB
_NVEC = _SEQ_PAD // 16

_mesh = plsc.VectorSubcoreMesh(core_axis_name="c", subcore_axis_name="s")


@functools.partial(
    pl.kernel,
    mesh=_mesh,
    compiler_params=pltpu.CompilerParams(use_tc_tiling_on_sc=False,
                                          needs_layout_passes=False),
    out_type=jax.ShapeDtypeStruct((_BATCH, _NUM_PT, _SEQ), jnp.float32),
    scratch_types=[
        pltpu.VMEM((_NUM_T,), jnp.float32),
        pltpu.VMEM((2, _NB, _SEQ_PAD), jnp.int32),
        pltpu.VMEM((2, _NB, _SEQ_PAD), jnp.float32),
        pltpu.SemaphoreType.DMA,
        pltpu.SemaphoreType.DMA,
        pltpu.SemaphoreType.DMA,
        pltpu.SemaphoreType.DMA,
    ],
)
def _sc_lookup(sent_hbm, rules_hbm, out_hbm, row_v, idx_v, outbuf_v,
               sem_in0, sem_in1, sem_out0, sem_out1):
    wid = lax.axis_index("s") * 2 + lax.axis_index("c")
    sem_in = (sem_in0, sem_in1)
    sem_out = (sem_out0, sem_out1)

    def in_copy(ci, b):
        return pltpu.make_async_copy(sent_hbm.at[pl.ds(ci * _NB, _NB)],
                                     idx_v.at[b], sem_in[b])

    def out_copy(ci, b):
        return pltpu.make_async_copy(outbuf_v.at[b, :, pl.ds(0, _SEQ)],
                                     out_hbm.at[pl.ds(ci * _NB, _NB), wid],
                                     sem_out[b])

    in_copy(0, 0).start()
    in_copy(1, 1).start()
    pltpu.sync_copy(rules_hbm.at[wid], row_v)

    def pair_body(p, _):
        for b in range(2):
            ci = p * 2 + b
            in_copy(ci, b).wait()

            @pl.when(p > 0)
            def _wait_out():
                out_copy(ci - 2, b).wait()

            def sent_body(s, _):
                for j in range(_NVEC):
                    idx = idx_v[b, s, pl.ds(j * 16, 16)]
                    outbuf_v[b, s, pl.ds(j * 16, 16)] = (
                        plsc.load_gather(row_v, [idx]))
                return 0

            lax.fori_loop(0, _NB, sent_body, 0)
            out_copy(ci, b).start()

            @pl.when(ci + 2 < _NCHUNK)
            def _prefetch():
                in_copy(ci + 2, b).start()
        return 0

    lax.fori_loop(0, _NCHUNK // 2, pair_body, 0)
    for b in range(2):
        out_copy(_NCHUNK - 2 + b, b).wait()


def kernel(sentences, rules):
    sent_pad = jnp.pad(sentences.astype(jnp.int32),
                       ((0, 0), (0, _SEQ_PAD - _SEQ)))
    return _sc_lookup(sent_pad, rules)


# speedup vs baseline: 4.5296x; 1.1090x over previous
import functools

import jax
import jax.numpy as jnp
from jax import lax
from jax.experimental import pallas as pl
from jax.experimental.pallas import tpu as pltpu
from jax.experimental.pallas import tpu_sc as plsc

_NUM_PT = 32
_NUM_T = 100000
_BATCH = 1024
_SEQ = 200
_SEQ_PAD = 208
_NB = 32
_NCHUNK = _BATCH // _NB
_NVEC = _SEQ_PAD // 16

_mesh = plsc.VectorSubcoreMesh(core_axis_name="c", subcore_axis_name="s")


@functools.partial(
    pl.kernel,
    mesh=_mesh,
    compiler_params=pltpu.CompilerParams(use_tc_tiling_on_sc=False,
                                          needs_layout_passes=False),
    out_type=jax.ShapeDtypeStruct((_BATCH, _NUM_PT, _SEQ), jnp.float32),
    scratch_types=[
        pltpu.VMEM((_NUM_T,), jnp.float32),
        pltpu.VMEM((2, _NB, _SEQ_PAD), jnp.int32),
        pltpu.VMEM((2, _NB, _SEQ_PAD), jnp.float32),
        pltpu.SemaphoreType.DMA,
        pltpu.SemaphoreType.DMA,
        pltpu.SemaphoreType.DMA,
        pltpu.SemaphoreType.DMA,
    ],
)
def _sc_lookup(sent_hbm, rules_hbm, out_hbm, row_v, idx_v, outbuf_v,
               sem_in0, sem_in1, sem_out0, sem_out1):
    wid = lax.axis_index("s") * 2 + lax.axis_index("c")
    sem_in = (sem_in0, sem_in1)
    sem_out = (sem_out0, sem_out1)

    def in_copy(ci, b):
        return pltpu.make_async_copy(sent_hbm.at[pl.ds(ci * _NB, _NB)],
                                     idx_v.at[b], sem_in[b])

    def out_copy(ci, b):
        return pltpu.make_async_copy(outbuf_v.at[b, :, pl.ds(0, _SEQ)],
                                     out_hbm.at[pl.ds(ci * _NB, _NB), wid],
                                     sem_out[b])

    in_copy(0, 0).start()
    in_copy(1, 1).start()
    pltpu.sync_copy(rules_hbm.at[wid], row_v)

    def pair_body(p, _):
        for b in range(2):
            ci = p * 2 + b
            in_copy(ci, b).wait()

            @pl.when(p > 0)
            def _wait_out():
                out_copy(ci - 2, b).wait()

            @plsc.parallel_loop(0, _NB, unroll=2)
            def sent_body(s):
                for j in range(_NVEC):
                    idx = idx_v[b, s, pl.ds(j * 16, 16)]
                    outbuf_v[b, s, pl.ds(j * 16, 16)] = (
                        plsc.load_gather(row_v, [idx]))
            out_copy(ci, b).start()

            @pl.when(ci + 2 < _NCHUNK)
            def _prefetch():
                in_copy(ci + 2, b).start()
        return 0

    lax.fori_loop(0, _NCHUNK // 2, pair_body, 0)
    for b in range(2):
        out_copy(_NCHUNK - 2 + b, b).wait()


def kernel(sentences, rules):
    sent_pad = jnp.pad(sentences.astype(jnp.int32),
                       ((0, 0), (0, _SEQ_PAD - _SEQ)))
    return _sc_lookup(sent_pad, rules)
